# Initial kernel scaffold; baseline (speedup 1.0000x reference)
import jax
import jax.numpy as jnp
from jax.experimental import pallas as pl


def kernel(query_points, support_points, neighbors_indices, features, K_values):
    raise NotImplementedError("write your pallas kernel here")


# speedup vs baseline: 1.3461x; 1.3461x over previous
import numpy as np
import jax
import jax.numpy as jnp
from jax.experimental import pallas as pl
from jax.experimental.pallas import tpu as pltpu

N = 10000
N0 = 10000
H = 32
DIM = 3
IN_FDIM = 128
OUT_FDIM = 128
K = 15
EXTENT = 1.0 * 2.5 / 5.0
NEG_SLOPE = 0.1

BN = 80
E = BN * H


def _kernel_points_np():
    rng = np.random.RandomState(42)
    dirs = rng.normal(size=(K - 1, DIM))
    dirs = dirs / (np.linalg.norm(dirs, axis=1, keepdims=True) + 1e-9)
    radii = rng.uniform(size=(K - 1, 1)) ** (1.0 / 3.0) * EXTENT
    return np.concatenate([np.zeros((1, DIM)), dirs * radii], axis=0).astype(np.float32)


_KP = _kernel_points_np()
_KP_T = np.ascontiguousarray(_KP.T)
_KP_SQ = np.sum(_KP * _KP, axis=1, keepdims=True).T.copy()


def _body(qp_ref, idx_ref, sp_ref, feat_ref, kv_ref, kpt_ref, kpsq_ref,
          out_ref, xg_ref, df_ref):
    def gather_b(b, carry):
        qrow = qp_ref[pl.ds(b, 1), :]
        for h in range(H):
            j = idx_ref[b, h]
            e = b * H + h
            xg_ref[pl.ds(e, 1), :] = feat_ref[pl.ds(j, 1), :]
            df_ref[pl.ds(e, 1), :] = sp_ref[pl.ds(j, 1), :] - qrow
        return carry

    jax.lax.fori_loop(0, BN, gather_b, 0)

    diff = df_ref[:, :]
    xg = xg_ref[:, :]
    dsq = jnp.sum(diff * diff, axis=1, keepdims=True)
    cross = jnp.dot(diff, kpt_ref[:, :],
                    preferred_element_type=jnp.float32)
    sq = dsq - 2.0 * cross + kpsq_ref[:, :]
    dist = jnp.sqrt(jnp.maximum(sq, 1e-12))
    w = jnp.maximum(1.0 - dist / EXTENT, 0.0)

    acc = jnp.zeros((BN, OUT_FDIM), jnp.float32)
    for k in range(K):
        wk = w[:, k:k + 1]
        wx = (xg * wk).reshape(BN, H, IN_FDIM).sum(axis=1)
        acc = acc + jnp.dot(wx, kv_ref[k],
                            preferred_element_type=jnp.float32)
    out_ref[:, :] = jnp.where(acc >= 0, acc, NEG_SLOPE * acc)


def kernel(query_points, support_points, neighbors_indices, features, K_values):
    kpt = jnp.asarray(_KP_T)
    kpsq = jnp.asarray(_KP_SQ)
    out = pl.pallas_call(
        _body,
        grid=(N // BN,),
        in_specs=[
            pl.BlockSpec((BN, DIM), lambda i: (i, 0)),
            pl.BlockSpec((BN, H), lambda i: (i, 0), memory_space=pltpu.SMEM),
            pl.BlockSpec((N0, DIM), lambda i: (0, 0)),
            pl.BlockSpec((N0, IN_FDIM), lambda i: (0, 0)),
            pl.BlockSpec((K, IN_FDIM, OUT_FDIM), lambda i: (0, 0, 0)),
            pl.BlockSpec((DIM, K), lambda i: (0, 0)),
            pl.BlockSpec((1, K), lambda i: (0, 0)),
        ],
        out_specs=pl.BlockSpec((BN, OUT_FDIM), lambda i: (i, 0)),
        out_shape=jax.ShapeDtypeStruct((N, OUT_FDIM), jnp.float32),
        scratch_shapes=[
            pltpu.VMEM((E, IN_FDIM), jnp.float32),
            pltpu.VMEM((E, DIM), jnp.float32),
        ],
    )(query_points, neighbors_indices, support_points, features, K_values,
      kpt, kpsq)
    return out


# speedup vs baseline: 2.5439x; 1.8899x over previous
import numpy as np
import jax
import jax.numpy as jnp
from jax.experimental import pallas as pl
from jax.experimental.pallas import tpu as pltpu

N = 10000
N0 = 10000
H = 32
DIM = 3
IN_FDIM = 128
OUT_FDIM = 128
K = 15
EXTENT = 1.0 * 2.5 / 5.0
NEG_SLOPE = 0.1

BN = 80
E = BN * H


def _kernel_points_np():
    rng = np.random.RandomState(42)
    dirs = rng.normal(size=(K - 1, DIM))
    dirs = dirs / (np.linalg.norm(dirs, axis=1, keepdims=True) + 1e-9)
    radii = rng.uniform(size=(K - 1, 1)) ** (1.0 / 3.0) * EXTENT
    return np.concatenate([np.zeros((1, DIM)), dirs * radii], axis=0).astype(np.float32)


_KP = _kernel_points_np()
_KP_T = np.ascontiguousarray(_KP.T)
_KP_SQ = np.sum(_KP * _KP, axis=1, keepdims=True).T.copy()


def _body(qp_ref, idx_ref, sp_ref, feat_ref, kv_ref, kpt_ref, kpsq_ref,
          out_ref, xg_ref, df_ref):
    def gather_b(b, carry):
        qrow = qp_ref[pl.ds(b, 1), :]
        for h in range(H):
            j = idx_ref[b, h]
            e = b * H + h
            xg_ref[pl.ds(e, 1), :] = feat_ref[pl.ds(j, 1), :]
            df_ref[pl.ds(e, 1), :] = sp_ref[pl.ds(j, 1), :] - qrow
        return carry

    jax.lax.fori_loop(0, BN, gather_b, 0)

    diff = df_ref[:, :]
    xg = xg_ref[:, :]
    dsq = jnp.sum(diff * diff, axis=1, keepdims=True)
    cross = jnp.dot(diff, kpt_ref[:, :],
                    preferred_element_type=jnp.float32)
    sq = dsq - 2.0 * cross + kpsq_ref[:, :]
    dist = jnp.sqrt(jnp.maximum(sq, 1e-12))
    w = jnp.maximum(1.0 - dist / EXTENT, 0.0)

    w3 = w.reshape(BN, H, K)
    xg3 = xg.reshape(BN, H, IN_FDIM)
    weighted = jax.lax.dot_general(
        w3, xg3, (((1,), (1,)), ((0,), (0,))),
        preferred_element_type=jnp.float32)
    acc = jnp.zeros((BN, OUT_FDIM), jnp.float32)
    for k in range(K):
        acc = acc + jnp.dot(weighted[:, k, :], kv_ref[k],
                            preferred_element_type=jnp.float32)
    out_ref[:, :] = jnp.where(acc >= 0, acc, NEG_SLOPE * acc)


def kernel(query_points, support_points, neighbors_indices, features, K_values):
    kpt = jnp.asarray(_KP_T)
    kpsq = jnp.asarray(_KP_SQ)
    out = pl.pallas_call(
        _body,
        grid=(N // BN,),
        in_specs=[
            pl.BlockSpec((BN, DIM), lambda i: (i, 0)),
            pl.BlockSpec((BN, H), lambda i: (i, 0), memory_space=pltpu.SMEM),
            pl.BlockSpec((N0, DIM), lambda i: (0, 0)),
            pl.BlockSpec((N0, IN_FDIM), lambda i: (0, 0)),
            pl.BlockSpec((K, IN_FDIM, OUT_FDIM), lambda i: (0, 0, 0)),
            pl.BlockSpec((DIM, K), lambda i: (0, 0)),
            pl.BlockSpec((1, K), lambda i: (0, 0)),
        ],
        out_specs=pl.BlockSpec((BN, OUT_FDIM), lambda i: (i, 0)),
        out_shape=jax.ShapeDtypeStruct((N, OUT_FDIM), jnp.float32),
        scratch_shapes=[
            pltpu.VMEM((E, IN_FDIM), jnp.float32),
            pltpu.VMEM((E, DIM), jnp.float32),
        ],
    )(query_points, neighbors_indices, support_points, features, K_values,
      kpt, kpsq)
    return out
